# baseline (speedup 1.0000x reference)
import functools
import jax
import jax.numpy as jnp
from jax import lax
from jax.experimental import pallas as pl
from jax.experimental.pallas import tpu as pltpu
from jax.experimental.pallas import tpu_sc as plsc

R, V = 128, 100000
NC, NS, L = 2, 16, 16
NW = NC * NS
ROWS_PER_W = R // NW
CHUNK = 20000
NCHUNK = V // CHUNK
UNROLL = 10
NACC = 5
INNER = CHUNK // (L * UNROLL)

NEG = -3.4e38


def _sc_max(x_hbm, out_hbm, buf0, buf1, out_buf, sem0, sem1):
    wid = lax.axis_index("s") * NC + lax.axis_index("c")
    row0 = wid * ROWS_PER_W
    bufs = (buf0, buf1)
    sems = (sem0, sem1)

    def issue(t):
        r, c = divmod(t, NCHUNK)
        off = pl.multiple_of((row0 + r) * V + c * CHUNK, 8)
        return pltpu.async_copy(
            x_hbm.at[pl.ds(off, CHUNK)],
            bufs[t % 2], sems[t % 2])

    T = ROWS_PER_W * NCHUNK
    cps = [issue(0), None]
    lane = lax.iota(jnp.int32, L)
    out_vec = jnp.full((L,), NEG, jnp.float32)
    accs = tuple(jnp.full((L,), NEG, jnp.float32) for _ in range(NACC))

    for t in range(T):
        if t + 1 < T:
            cps[(t + 1) % 2] = issue(t + 1)
        cps[t % 2].wait()
        buf = bufs[t % 2]

        def body(i, a, buf=buf):
            base = i * (L * UNROLL)
            out = list(a)
            for u in range(UNROLL):
                j = u % NACC
                out[j] = jnp.maximum(out[j], buf[pl.ds(base + u * L, L)])
            return tuple(out)

        accs = lax.fori_loop(0, INNER, body, accs)

        r, c = divmod(t, NCHUNK)
        if c == NCHUNK - 1:
            a = accs[0]
            for j in range(1, NACC):
                a = jnp.maximum(a, accs[j])
            m = jnp.max(a)
            out_vec = jnp.where(lane == r, m, out_vec)
            accs = tuple(jnp.full((L,), NEG, jnp.float32) for _ in range(NACC))

    out_buf[...] = out_vec
    pltpu.sync_copy(out_buf, out_hbm.at[wid])


def kernel(X):
    call = pl.kernel(
        _sc_max,
        out_type=jax.ShapeDtypeStruct((NW, L), jnp.float32),
        mesh=plsc.VectorSubcoreMesh(core_axis_name="c", subcore_axis_name="s"),
        scratch_types=[
            pltpu.VMEM((CHUNK,), jnp.float32),
            pltpu.VMEM((CHUNK,), jnp.float32),
            pltpu.VMEM((L,), jnp.float32),
            pltpu.SemaphoreType.DMA,
            pltpu.SemaphoreType.DMA,
        ],
        compiler_params=pltpu.CompilerParams(needs_layout_passes=False),
    )
    out2 = call(X.reshape(-1))
    return out2[:, :ROWS_PER_W].reshape(R)


# speedup vs baseline: 2.4142x; 2.4142x over previous
import jax
import jax.numpy as jnp
from jax.experimental import pallas as pl
from jax.experimental.pallas import tpu as pltpu

R, V = 128, 100000
RB = 8
T = R // RB
K = 6


def _max_body(x_hbm, o_ref, *scratch):
    bufs = scratch[:K]
    sems = scratch[K:]

    def issue(t):
        return pltpu.make_async_copy(
            x_hbm.at[pl.ds(t * RB, RB), :], bufs[t % K], sems[t % K])

    cps = [issue(t) for t in range(K)]
    for cp in cps:
        cp.start()
    for t in range(T):
        cps[t % K].wait()
        o_ref[pl.ds(t * RB, RB), 0] = jnp.max(bufs[t % K][...], axis=-1)
        if t + K < T:
            cps[t % K] = issue(t + K)
            cps[t % K].start()


def kernel(X):
    out = pl.pallas_call(
        _max_body,
        in_specs=[pl.BlockSpec(memory_space=pl.ANY)],
        out_specs=pl.BlockSpec(memory_space=pltpu.MemorySpace.VMEM),
        out_shape=jax.ShapeDtypeStruct((R, 1), jnp.float32),
        scratch_shapes=(
            [pltpu.VMEM((RB, V), jnp.float32) for _ in range(K)]
            + [pltpu.SemaphoreType.DMA for _ in range(K)]
        ),
    )(X)
    return out[:, 0]


# speedup vs baseline: 9.0204x; 3.7364x over previous
import jax
import jax.numpy as jnp
from jax.experimental import pallas as pl
from jax.experimental.pallas import tpu as pltpu

R, V = 128, 100000
T = 20
CR = V // T
K = 6

NEG = -3.4e38


def _max_body(xt_hbm, o_ref, *scratch):
    bufs = scratch[:K]
    sems = scratch[K:]

    def issue(t):
        return pltpu.make_async_copy(
            xt_hbm.at[pl.ds(t * CR, CR), :], bufs[t % K], sems[t % K])

    cps = [issue(t) for t in range(K)]
    for cp in cps:
        cp.start()
    acc = jnp.full((R,), NEG, jnp.float32)
    for t in range(T):
        cps[t % K].wait()
        acc = jnp.maximum(acc, jnp.max(bufs[t % K][...], axis=0))
        if t + K < T:
            cps[t % K] = issue(t + K)
            cps[t % K].start()
    o_ref[0, :] = acc


def kernel(X):
    out = pl.pallas_call(
        _max_body,
        in_specs=[pl.BlockSpec(memory_space=pl.ANY)],
        out_specs=pl.BlockSpec(memory_space=pltpu.MemorySpace.VMEM),
        out_shape=jax.ShapeDtypeStruct((1, R), jnp.float32),
        scratch_shapes=(
            [pltpu.VMEM((CR, R), jnp.float32) for _ in range(K)]
            + [pltpu.SemaphoreType.DMA for _ in range(K)]
        ),
    )(X.T)
    return out[0]
